# baseline (speedup 1.0000x reference)
import functools

import jax
import jax.numpy as jnp
from jax import lax
from jax.experimental import pallas as pl
from jax.experimental.pallas import tpu as pltpu
from jax.experimental.pallas import tpu_sc as plsc

_B, _H, _D = 16384, 50, 64
_V = 1000000
_NC, _NS = 2, 16
_NW = _NC * _NS
_CBW = _B // 128 // _NW
_UNITS = _CBW * _H
_CB = 8192


def _tc_transpose_body(x_ref, o_ref):
    xt = x_ref[...].T.reshape(_CB // 2, 2, 64)
    o_ref[...] = jnp.concatenate([xt[:, 0, :], xt[:, 1, :]], axis=1)


def _linearize_table(table):
    table_t = table.T
    grid = (_V + _CB - 1) // _CB
    lin = pl.pallas_call(
        _tc_transpose_body,
        grid=(grid,),
        in_specs=[pl.BlockSpec((64, _CB), lambda i: (0, i))],
        out_specs=pl.BlockSpec((_CB // 2, 128), lambda i: (i, 0)),
        out_shape=jax.ShapeDtypeStruct((_V // 2, 128), jnp.float32),
    )(table_t)
    return lin.reshape(_V, _D)


def _sc_body(idx_hbm, table_hbm, out_hbm, idx_v, rows_v, otile_v,
             gsem0, gsem1, osem0, osem1):
    gsems = (gsem0, gsem1)
    osems = (osem0, osem1)
    wid = lax.axis_index("s") * _NC + lax.axis_index("c")

    pltpu.sync_copy(idx_hbm.at[:, pl.ds(wid * _CBW, _CBW), :], idx_v)

    lanes = [lax.iota(jnp.int32, 16) + 16 * q for q in range(8)]

    def unit_hc(u):
        return u // _H, lax.rem(u, _H)

    def issue_gather(u, b):
        cc, h = unit_hc(u)
        pltpu.async_copy(table_hbm.at[idx_v.at[h, cc]], rows_v.at[b], gsems[b])

    def wait_gather(b):
        pltpu.make_async_copy(
            table_hbm.at[pl.ds(0, 128)], rows_v.at[b], gsems[b]
        ).wait()

    def transpose_unit(b):
        rows = rows_v.at[b]
        otile = otile_v.at[b]

        def dbody(d, carry):
            col = jnp.full((16,), d, jnp.int32)
            for q in range(8):
                v = plsc.load_gather(rows, [lanes[q], col])
                otile[d, pl.ds(16 * q, 16)] = v
            return carry

        lax.fori_loop(0, _D, dbody, 0, unroll=False)

    def issue_writes(u, b):
        cc, h = unit_hc(u)
        tbase = h * 1024 + wid * _CBW + cc
        for dh in range(8):
            pltpu.async_copy(
                otile_v.at[b, pl.ds(8 * dh, 8), :],
                out_hbm.at[tbase + dh * 128],
                osems[b],
            )

    def wait_writes(b):
        for dh in range(8):
            pltpu.make_async_copy(
                out_hbm.at[0], otile_v.at[b, pl.ds(8 * dh, 8), :], osems[b]
            ).wait()

    issue_gather(0, 0)

    def outer(gi, carry):
        for s in range(2):
            u = 2 * gi + s
            wait_gather(s)

            @pl.when(u < _UNITS - 1)
            def _():
                issue_gather(u + 1, 1 - s)

            @pl.when(gi >= 1)
            def _():
                wait_writes(s)

            transpose_unit(s)
            issue_writes(u, s)
        return carry

    lax.fori_loop(0, _UNITS // 2, outer, 0, unroll=False)
    wait_writes(0)
    wait_writes(1)


def kernel(inp, table):
    table_lin = _linearize_table(table)
    idx3 = inp.astype(jnp.int32).T.reshape(_H, _B // 128, 128)
    mesh = plsc.VectorSubcoreMesh(core_axis_name="c", subcore_axis_name="s")
    out = pl.kernel(
        _sc_body,
        out_type=jax.ShapeDtypeStruct((_H * 8 * 128, 8, 128), jnp.float32),
        mesh=mesh,
        compiler_params=pltpu.CompilerParams(
            use_tc_tiling_on_sc=False, needs_layout_passes=False),
        scratch_types=[
            pltpu.VMEM((_H, _CBW, 128), jnp.int32),
            pltpu.VMEM((2, 128, _D), jnp.float32),
            pltpu.VMEM((2, _D, 128), jnp.float32),
            pltpu.SemaphoreType.DMA,
            pltpu.SemaphoreType.DMA,
            pltpu.SemaphoreType.DMA,
            pltpu.SemaphoreType.DMA,
        ],
    )(idx3, table_lin)
    flat = out.reshape(_B * _H * _D)
    res = flat.reshape(_H, 8, 128, 8, 128).transpose(2, 4, 0, 1, 3)
    return res.reshape(_B, _H, _D)


# speedup vs baseline: 1.4576x; 1.4576x over previous
import functools

import jax
import jax.numpy as jnp
from jax import lax
from jax.experimental import pallas as pl
from jax.experimental.pallas import tpu as pltpu
from jax.experimental.pallas import tpu_sc as plsc

_B, _H, _D = 16384, 50, 64
_V = 1000000
_NC, _NS = 2, 16
_NW = _NC * _NS
_CBW = _B // 128 // _NW
_UNITS = _CBW * _H
_CB = 8192


def _tc_transpose_body(x_ref, o_ref):
    xt = x_ref[...].T.reshape(_CB // 2, 2, 64)
    o_ref[...] = jnp.concatenate([xt[:, 0, :], xt[:, 1, :]], axis=1)


def _linearize_table(table):
    table_t = table.T
    grid = (_V + _CB - 1) // _CB
    lin = pl.pallas_call(
        _tc_transpose_body,
        grid=(grid,),
        in_specs=[pl.BlockSpec((64, _CB), lambda i: (0, i))],
        out_specs=pl.BlockSpec((_CB // 2, 128), lambda i: (i, 0)),
        out_shape=jax.ShapeDtypeStruct((_V // 2, 128), jnp.float32),
    )(table_t)
    return lin.reshape(_V, _D)


def _sc_body(idx_hbm, table_hbm, out_hbm, idx_v, rows_v, otile_v,
             gsem0, gsem1, osem0, osem1):
    gsems = (gsem0, gsem1)
    osems = (osem0, osem1)
    wid = lax.axis_index("s") * _NC + lax.axis_index("c")

    pltpu.sync_copy(idx_hbm.at[:, pl.ds(wid * _CBW, _CBW), :], idx_v)

    lanes = [lax.iota(jnp.int32, 16) + 16 * q for q in range(8)]

    def unit_hc(u):
        return u // _H, lax.rem(u, _H)

    def issue_gather(u, b):
        cc, h = unit_hc(u)
        pltpu.async_copy(table_hbm.at[idx_v.at[h, cc]], rows_v.at[b], gsems[b])

    def wait_gather(b):
        pltpu.make_async_copy(
            table_hbm.at[pl.ds(0, 128)], rows_v.at[b], gsems[b]
        ).wait()

    def transpose_unit(b):
        rows = rows_v.at[b]
        otile = otile_v.at[b]

        @plsc.parallel_loop(0, _D, unroll=8)
        def dbody(d):
            col = jnp.full((16,), d, jnp.int32)
            for q in range(8):
                v = plsc.load_gather(rows, [lanes[q], col])
                otile[d, pl.ds(16 * q, 16)] = v

    def issue_writes(u, b):
        cc, h = unit_hc(u)
        tbase = h * 1024 + wid * _CBW + cc
        for dh in range(8):
            pltpu.async_copy(
                otile_v.at[b, pl.ds(8 * dh, 8), :],
                out_hbm.at[tbase + dh * 128],
                osems[b],
            )

    def wait_writes(b):
        for dh in range(8):
            pltpu.make_async_copy(
                out_hbm.at[0], otile_v.at[b, pl.ds(8 * dh, 8), :], osems[b]
            ).wait()

    issue_gather(0, 0)

    def outer(gi, carry):
        for s in range(2):
            u = 2 * gi + s
            wait_gather(s)

            @pl.when(u < _UNITS - 1)
            def _():
                issue_gather(u + 1, 1 - s)

            @pl.when(gi >= 1)
            def _():
                wait_writes(s)

            transpose_unit(s)
            issue_writes(u, s)
        return carry

    lax.fori_loop(0, _UNITS // 2, outer, 0, unroll=False)
    wait_writes(0)
    wait_writes(1)


def kernel(inp, table):
    table_lin = _linearize_table(table)
    idx3 = inp.astype(jnp.int32).T.reshape(_H, _B // 128, 128)
    mesh = plsc.VectorSubcoreMesh(core_axis_name="c", subcore_axis_name="s")
    out = pl.kernel(
        _sc_body,
        out_type=jax.ShapeDtypeStruct((_H * 8 * 128, 8, 128), jnp.float32),
        mesh=mesh,
        compiler_params=pltpu.CompilerParams(
            use_tc_tiling_on_sc=False, needs_layout_passes=False),
        scratch_types=[
            pltpu.VMEM((_H, _CBW, 128), jnp.int32),
            pltpu.VMEM((2, 128, _D), jnp.float32),
            pltpu.VMEM((2, _D, 128), jnp.float32),
            pltpu.SemaphoreType.DMA,
            pltpu.SemaphoreType.DMA,
            pltpu.SemaphoreType.DMA,
            pltpu.SemaphoreType.DMA,
        ],
    )(idx3, table_lin)
    flat = out.reshape(_B * _H * _D)
    res = flat.reshape(_H, 8, 128, 8, 128).transpose(2, 4, 0, 1, 3)
    return res.reshape(_B, _H, _D)


# speedup vs baseline: 1.4917x; 1.0234x over previous
import functools

import jax
import jax.numpy as jnp
from jax import lax
from jax.experimental import pallas as pl
from jax.experimental.pallas import tpu as pltpu
from jax.experimental.pallas import tpu_sc as plsc

_B, _H, _D = 16384, 50, 64
_V = 1000000
_NC, _NS = 2, 16
_NW = _NC * _NS
_CBW = _B // 128 // _NW
_UNITS = _CBW * _H
_CB = 16384


def _tc_transpose_body(x_ref, o_ref):
    xt = x_ref[...].T.reshape(_CB // 2, 2, 64)
    o_ref[...] = jnp.concatenate([xt[:, 0, :], xt[:, 1, :]], axis=1)


def _linearize_table(table):
    table_t = table.T
    grid = (_V + _CB - 1) // _CB
    lin = pl.pallas_call(
        _tc_transpose_body,
        grid=(grid,),
        in_specs=[pl.BlockSpec((64, _CB), lambda i: (0, i))],
        out_specs=pl.BlockSpec((_CB // 2, 128), lambda i: (i, 0)),
        out_shape=jax.ShapeDtypeStruct((_V // 2, 128), jnp.float32),
    )(table_t)
    return lin.reshape(_V, _D)


def _sc_body(idx_hbm, table_hbm, out_hbm, idx_v, rows_v, otile_v,
             gsem0, gsem1, osem0, osem1):
    gsems = (gsem0, gsem1)
    osems = (osem0, osem1)
    wid = lax.axis_index("s") * _NC + lax.axis_index("c")

    pltpu.sync_copy(idx_hbm.at[:, pl.ds(wid * _CBW, _CBW), :], idx_v)

    lanes = [lax.iota(jnp.int32, 16) + 16 * q for q in range(8)]

    def unit_hc(u):
        return u // _H, lax.rem(u, _H)

    def issue_gather(u, b):
        cc, h = unit_hc(u)
        pltpu.async_copy(table_hbm.at[idx_v.at[h, cc]], rows_v.at[b], gsems[b])

    def wait_gather(b):
        pltpu.make_async_copy(
            table_hbm.at[pl.ds(0, 128)], rows_v.at[b], gsems[b]
        ).wait()

    def transpose_unit(b):
        rows = rows_v.at[b]
        otile = otile_v.at[b]

        @plsc.parallel_loop(0, _D, unroll=16)
        def dbody(d):
            col = jnp.full((16,), d, jnp.int32)
            for q in range(8):
                v = plsc.load_gather(rows, [lanes[q], col])
                otile[d, pl.ds(16 * q, 16)] = v

    def issue_writes(u, b):
        cc, h = unit_hc(u)
        tbase = h * 1024 + wid * _CBW + cc
        for dh in range(8):
            pltpu.async_copy(
                otile_v.at[b, pl.ds(8 * dh, 8), :],
                out_hbm.at[tbase + dh * 128],
                osems[b],
            )

    def wait_writes(b):
        for dh in range(8):
            pltpu.make_async_copy(
                out_hbm.at[0], otile_v.at[b, pl.ds(8 * dh, 8), :], osems[b]
            ).wait()

    issue_gather(0, 0)

    def outer(gi, carry):
        for s in range(2):
            u = 2 * gi + s
            wait_gather(s)

            @pl.when(u < _UNITS - 1)
            def _():
                issue_gather(u + 1, 1 - s)

            @pl.when(gi >= 1)
            def _():
                wait_writes(s)

            transpose_unit(s)
            issue_writes(u, s)
        return carry

    lax.fori_loop(0, _UNITS // 2, outer, 0, unroll=False)
    wait_writes(0)
    wait_writes(1)


def kernel(inp, table):
    table_lin = _linearize_table(table)
    idx3 = inp.astype(jnp.int32).T.reshape(_H, _B // 128, 128)
    mesh = plsc.VectorSubcoreMesh(core_axis_name="c", subcore_axis_name="s")
    out = pl.kernel(
        _sc_body,
        out_type=jax.ShapeDtypeStruct((_H * 8 * 128, 8, 128), jnp.float32),
        mesh=mesh,
        compiler_params=pltpu.CompilerParams(
            use_tc_tiling_on_sc=False, needs_layout_passes=False),
        scratch_types=[
            pltpu.VMEM((_H, _CBW, 128), jnp.int32),
            pltpu.VMEM((2, 128, _D), jnp.float32),
            pltpu.VMEM((2, _D, 128), jnp.float32),
            pltpu.SemaphoreType.DMA,
            pltpu.SemaphoreType.DMA,
            pltpu.SemaphoreType.DMA,
            pltpu.SemaphoreType.DMA,
        ],
    )(idx3, table_lin)
    flat = out.reshape(_B * _H * _D)
    res = flat.reshape(_H, 8, 128, 8, 128).transpose(2, 4, 0, 1, 3)
    return res.reshape(_B, _H, _D)
